# Initial kernel scaffold; baseline (speedup 1.0000x reference)
import jax
import jax.numpy as jnp
from jax.experimental import pallas as pl


def kernel(x, edge_index, batch, W0, b0, g0, be0, L0W, L0b, W1, b1, g1, be1, L1W, L1b):
    raise NotImplementedError("write your pallas kernel here")


# speedup vs baseline: 7.8886x; 7.8886x over previous
import functools

import jax
import jax.numpy as jnp
from jax import lax
from jax.experimental import pallas as pl
from jax.experimental.pallas import tpu as pltpu
from jax.experimental.pallas import tpu_sc as plsc

N = 10000
E = 320000
DF = 128
H = 64
B = 8

BLK = 1000
NBLK = N // BLK

NW = 32
EPW = E // NW
CHUNK = 80
NCH = EPW // CHUNK
NTILE = 16
STRIPE = 624
TAIL = N - NTILE * STRIPE


def _tc1_body(x_ref, batch_ref, w0_ref, a0_ref, c0_ref, l0_ref, l0b_ref,
              feat_ref, np0_ref, wsi_ref):
    i = pl.program_id(0)
    h = jnp.dot(x_ref[...].astype(jnp.bfloat16),
                w0_ref[...].astype(jnp.bfloat16),
                preferred_element_type=jnp.float32)
    feat = jnp.maximum(h * a0_ref[...] + c0_ref[...], 0.0)
    feat_ref[...] = feat
    np0 = jnp.dot(feat.astype(jnp.bfloat16), l0_ref[...].astype(jnp.bfloat16),
                  preferred_element_type=jnp.float32) + l0b_ref[0, 0]
    np0_ref[...] = np0
    seg = lax.broadcasted_iota(jnp.int32, (BLK, B), 1)
    vals = jnp.where(batch_ref[...] == seg, np0, -jnp.inf)
    m8 = jnp.max(vals, axis=0).reshape(1, B)
    prev = jnp.where(i == 0, jnp.full((1, B), -jnp.inf), wsi_ref[...])
    wsi_ref[...] = jnp.maximum(prev, m8)


def _make_tc1(interpret=False):
    return pl.pallas_call(
        _tc1_body,
        interpret=interpret,
        grid=(NBLK,),
    in_specs=[
        pl.BlockSpec((BLK, DF), lambda i: (i, 0)),
        pl.BlockSpec((BLK, 1), lambda i: (i, 0)),
        pl.BlockSpec((DF, H), lambda i: (0, 0)),
        pl.BlockSpec((1, H), lambda i: (0, 0)),
        pl.BlockSpec((1, H), lambda i: (0, 0)),
        pl.BlockSpec((H, 1), lambda i: (0, 0)),
        pl.BlockSpec(memory_space=pltpu.SMEM),
    ],
    out_specs=[
        pl.BlockSpec((BLK, H), lambda i: (i, 0)),
        pl.BlockSpec((BLK, 1), lambda i: (i, 0)),
        pl.BlockSpec((1, B), lambda i: (0, 0)),
    ],
    out_shape=[
        jax.ShapeDtypeStruct((N, H), jnp.float32),
        jax.ShapeDtypeStruct((N, 1), jnp.float32),
        jax.ShapeDtypeStruct((1, B), jnp.float32),
    ],
    )


def _tc2_body(feat_ref, agg_ref, batch_ref, np0_ref, wsi0_ref, w1_ref,
              a1_ref, c1_ref, l1_ref, l1b_ref, np_ref, wsi_ref):
    i = pl.program_id(0)
    h = feat_ref[...] + agg_ref[0] + agg_ref[1]
    z = jnp.dot(h.astype(jnp.bfloat16), w1_ref[...].astype(jnp.bfloat16),
                preferred_element_type=jnp.float32)
    feat2 = jnp.maximum(z * a1_ref[...] + c1_ref[...], 0.0)
    np1 = jnp.dot(feat2.astype(jnp.bfloat16), l1_ref[...].astype(jnp.bfloat16),
                  preferred_element_type=jnp.float32) + l1b_ref[0, 0]
    np_ref[...] = np0_ref[...] + np1
    seg = lax.broadcasted_iota(jnp.int32, (BLK, B), 1)
    vals = jnp.where(batch_ref[...] == seg, np1, -jnp.inf)
    m8 = jnp.max(vals, axis=0).reshape(1, B)
    prev = jnp.where(i == 0, jnp.full((1, B), -jnp.inf), wsi_ref[...])
    acc = jnp.maximum(prev, m8)
    wsi_ref[...] = acc

    @pl.when(i == NBLK - 1)
    def _():
        wsi_ref[...] = acc + wsi0_ref[...]


def _make_tc2(interpret=False):
    return pl.pallas_call(
        _tc2_body,
        interpret=interpret,
        grid=(NBLK,),
    in_specs=[
        pl.BlockSpec((BLK, H), lambda i: (i, 0)),
        pl.BlockSpec((2, BLK, H), lambda i: (0, i, 0)),
        pl.BlockSpec((BLK, 1), lambda i: (i, 0)),
        pl.BlockSpec((BLK, 1), lambda i: (i, 0)),
        pl.BlockSpec((1, B), lambda i: (0, 0)),
        pl.BlockSpec((H, H), lambda i: (0, 0)),
        pl.BlockSpec((1, H), lambda i: (0, 0)),
        pl.BlockSpec((1, H), lambda i: (0, 0)),
        pl.BlockSpec((H, 1), lambda i: (0, 0)),
        pl.BlockSpec(memory_space=pltpu.SMEM),
    ],
    out_specs=[
        pl.BlockSpec((BLK, 1), lambda i: (i, 0)),
        pl.BlockSpec((1, B), lambda i: (0, 0)),
    ],
    out_shape=[
        jax.ShapeDtypeStruct((N, 1), jnp.float32),
        jax.ShapeDtypeStruct((1, B), jnp.float32),
    ],
    )


_tc1 = _make_tc1()
_tc2 = _make_tc2()


def _sc_body(feat_hbm, src_hbm, dst_hbm, zer_hbm, out_hbm,
             src_v, dst_v, rows_v, zbuf, agg_sh, sem):
    cid = lax.axis_index("c")
    sid = lax.axis_index("s")
    wid = sid * 2 + cid

    pltpu.sync_copy(zer_hbm, zbuf)
    pltpu.sync_copy(zbuf, agg_sh.at[pl.ds(sid * STRIPE, STRIPE)])

    @pl.when(sid == 0)
    def _():
        pltpu.sync_copy(zbuf.at[pl.ds(0, TAIL)],
                        agg_sh.at[pl.ds(NTILE * STRIPE, TAIL)])

    plsc.subcore_barrier()

    pltpu.sync_copy(src_hbm.at[wid], src_v)
    pltpu.sync_copy(dst_hbm.at[wid], dst_v)

    def step(c, _):
        pltpu.async_copy(feat_hbm.at[src_v.at[c]], rows_v, sem).wait()
        pltpu.sync_copy(rows_v, agg_sh.at[dst_v.at[c]], add=True)
        return 0

    lax.fori_loop(0, NCH, step, 0)
    plsc.subcore_barrier()

    pltpu.sync_copy(agg_sh.at[pl.ds(sid * STRIPE, STRIPE)], zbuf)
    pltpu.sync_copy(zbuf, out_hbm.at[cid, pl.ds(sid * STRIPE, STRIPE)])

    @pl.when(sid == 0)
    def _():
        pltpu.sync_copy(agg_sh.at[pl.ds(NTILE * STRIPE, TAIL)],
                        zbuf.at[pl.ds(0, TAIL)])
        pltpu.sync_copy(zbuf.at[pl.ds(0, TAIL)],
                        out_hbm.at[cid, pl.ds(NTILE * STRIPE, TAIL)])


@functools.lru_cache(maxsize=1)
def _sc_seg_sum():
    return pl.kernel(
        _sc_body,
        out_type=jax.ShapeDtypeStruct((2, N, H), jnp.float32),
        mesh=plsc.VectorSubcoreMesh(core_axis_name="c", subcore_axis_name="s"),
        scratch_types=[
            pltpu.VMEM((NCH, CHUNK), jnp.int32),
            pltpu.VMEM((NCH, CHUNK), jnp.int32),
            pltpu.VMEM((CHUNK, H), jnp.float32),
            pltpu.VMEM((STRIPE, H), jnp.float32),
            pltpu.VMEM_SHARED((N, H), jnp.float32),
            pltpu.SemaphoreType.DMA,
        ],
        compiler_params=pltpu.CompilerParams(use_tc_tiling_on_sc=False),
    )


def kernel(x, edge_index, batch, W0, b0, g0, be0, L0W, L0b, W1, b1, g1, be1,
           L1W, L1b):
    s = jnp.float32(1.0) / jnp.sqrt(jnp.float32(1.0 + 1e-5))
    a0 = (g0 * s).reshape(1, H)
    c0 = (be0 + b0 * g0 * s).reshape(1, H)
    a1 = (g1 * s).reshape(1, H)
    c1 = (be1 + b1 * g1 * s).reshape(1, H)
    l0 = L0W
    l1 = L1W
    l0b = L0b.reshape(1, 1)
    l1b = L1b.reshape(1, 1)
    batch2 = batch.reshape(N, 1)

    feat, np0, wsi0 = _tc1(x, batch2, W0, a0, c0, l0, l0b)

    srcr = edge_index[0].reshape(NW, NCH, CHUNK)
    dstr = edge_index[1].reshape(NW, NCH, CHUNK)
    zer = jnp.zeros((STRIPE, H), jnp.float32)
    agg2 = _sc_seg_sum()(feat, srcr, dstr, zer)

    npred, wsi = _tc2(feat, agg2, batch2, np0, wsi0, W1, a1, c1, l1, l1b)
    return (wsi.reshape(B, 1), npred)


# speedup vs baseline: 9.0380x; 1.1457x over previous
import functools

import jax
import jax.numpy as jnp
from jax import lax
from jax.experimental import pallas as pl
from jax.experimental.pallas import tpu as pltpu
from jax.experimental.pallas import tpu_sc as plsc

N = 10000
E = 320000
DF = 128
H = 64
B = 8

BLK = 1000
NBLK = N // BLK

NW = 32
EPW = E // NW
CHUNK = 80
NCH = EPW // CHUNK
NTILE = 16
STRIPE = 624
TAIL = N - NTILE * STRIPE


def _tc1_body(x_ref, batch_ref, w0_ref, a0_ref, c0_ref, l0_ref, l0b_ref,
              feat_ref, np0_ref, wsi_ref):
    i = pl.program_id(0)
    h = jnp.dot(x_ref[...].astype(jnp.bfloat16),
                w0_ref[...].astype(jnp.bfloat16),
                preferred_element_type=jnp.float32)
    feat = jnp.maximum(h * a0_ref[...] + c0_ref[...], 0.0)
    feat_ref[...] = feat
    np0 = jnp.dot(feat.astype(jnp.bfloat16), l0_ref[...].astype(jnp.bfloat16),
                  preferred_element_type=jnp.float32) + l0b_ref[0, 0]
    np0_ref[...] = np0
    seg = lax.broadcasted_iota(jnp.int32, (BLK, B), 1)
    vals = jnp.where(batch_ref[...] == seg, np0, -jnp.inf)
    m8 = jnp.max(vals, axis=0).reshape(1, B)
    prev = jnp.where(i == 0, jnp.full((1, B), -jnp.inf), wsi_ref[...])
    wsi_ref[...] = jnp.maximum(prev, m8)


def _make_tc1(interpret=False):
    return pl.pallas_call(
        _tc1_body,
        interpret=interpret,
        grid=(NBLK,),
    in_specs=[
        pl.BlockSpec((BLK, DF), lambda i: (i, 0)),
        pl.BlockSpec((BLK, 1), lambda i: (i, 0)),
        pl.BlockSpec((DF, H), lambda i: (0, 0)),
        pl.BlockSpec((1, H), lambda i: (0, 0)),
        pl.BlockSpec((1, H), lambda i: (0, 0)),
        pl.BlockSpec((H, 1), lambda i: (0, 0)),
        pl.BlockSpec(memory_space=pltpu.SMEM),
    ],
    out_specs=[
        pl.BlockSpec((BLK, H), lambda i: (i, 0)),
        pl.BlockSpec((BLK, 1), lambda i: (i, 0)),
        pl.BlockSpec((1, B), lambda i: (0, 0)),
    ],
    out_shape=[
        jax.ShapeDtypeStruct((N, H), jnp.float32),
        jax.ShapeDtypeStruct((N, 1), jnp.float32),
        jax.ShapeDtypeStruct((1, B), jnp.float32),
    ],
    )


def _tc2_body(feat_ref, agg_ref, batch_ref, np0_ref, wsi0_ref, w1_ref,
              a1_ref, c1_ref, l1_ref, l1b_ref, np_ref, wsi_ref):
    i = pl.program_id(0)
    h = feat_ref[...] + agg_ref[0] + agg_ref[1]
    z = jnp.dot(h.astype(jnp.bfloat16), w1_ref[...].astype(jnp.bfloat16),
                preferred_element_type=jnp.float32)
    feat2 = jnp.maximum(z * a1_ref[...] + c1_ref[...], 0.0)
    np1 = jnp.dot(feat2.astype(jnp.bfloat16), l1_ref[...].astype(jnp.bfloat16),
                  preferred_element_type=jnp.float32) + l1b_ref[0, 0]
    np_ref[...] = np0_ref[...] + np1
    seg = lax.broadcasted_iota(jnp.int32, (BLK, B), 1)
    vals = jnp.where(batch_ref[...] == seg, np1, -jnp.inf)
    m8 = jnp.max(vals, axis=0).reshape(1, B)
    prev = jnp.where(i == 0, jnp.full((1, B), -jnp.inf), wsi_ref[...])
    acc = jnp.maximum(prev, m8)
    wsi_ref[...] = acc

    @pl.when(i == NBLK - 1)
    def _():
        wsi_ref[...] = acc + wsi0_ref[...]


def _make_tc2(interpret=False):
    return pl.pallas_call(
        _tc2_body,
        interpret=interpret,
        grid=(NBLK,),
    in_specs=[
        pl.BlockSpec((BLK, H), lambda i: (i, 0)),
        pl.BlockSpec((2, BLK, H), lambda i: (0, i, 0)),
        pl.BlockSpec((BLK, 1), lambda i: (i, 0)),
        pl.BlockSpec((BLK, 1), lambda i: (i, 0)),
        pl.BlockSpec((1, B), lambda i: (0, 0)),
        pl.BlockSpec((H, H), lambda i: (0, 0)),
        pl.BlockSpec((1, H), lambda i: (0, 0)),
        pl.BlockSpec((1, H), lambda i: (0, 0)),
        pl.BlockSpec((H, 1), lambda i: (0, 0)),
        pl.BlockSpec(memory_space=pltpu.SMEM),
    ],
    out_specs=[
        pl.BlockSpec((BLK, 1), lambda i: (i, 0)),
        pl.BlockSpec((1, B), lambda i: (0, 0)),
    ],
    out_shape=[
        jax.ShapeDtypeStruct((N, 1), jnp.float32),
        jax.ShapeDtypeStruct((1, B), jnp.float32),
    ],
    )


_tc1 = _make_tc1()
_tc2 = _make_tc2()


def _sc_body(feat_hbm, src_hbm, dst_hbm, zer_hbm, out_hbm,
             src_v, dst_v, rows_a, rows_b, zbuf, agg_sh, sem_a, sem_b):
    cid = lax.axis_index("c")
    sid = lax.axis_index("s")
    wid = sid * 2 + cid

    pltpu.sync_copy(zer_hbm, zbuf)
    pltpu.sync_copy(zbuf, agg_sh.at[pl.ds(sid * STRIPE, STRIPE)])

    @pl.when(sid == 0)
    def _():
        pltpu.sync_copy(zbuf.at[pl.ds(0, TAIL)],
                        agg_sh.at[pl.ds(NTILE * STRIPE, TAIL)])

    plsc.subcore_barrier()

    pltpu.sync_copy(src_hbm.at[wid], src_v)
    pltpu.sync_copy(dst_hbm.at[wid], dst_v)

    bufs = ((rows_a, sem_a), (rows_b, sem_b))
    pltpu.async_copy(feat_hbm.at[src_v.at[0]], rows_a, sem_a)

    def step2(k, _):
        for b in range(2):
            c = 2 * k + b
            cur_rows, cur_sem = bufs[b]
            nxt_rows, nxt_sem = bufs[1 - b]
            pltpu.make_async_copy(feat_hbm.at[src_v.at[c]], cur_rows,
                                  cur_sem).wait()

            @pl.when(c + 1 < NCH)
            def _():
                pltpu.async_copy(feat_hbm.at[src_v.at[c + 1]], nxt_rows,
                                 nxt_sem)

            pltpu.sync_copy(cur_rows, agg_sh.at[dst_v.at[c]], add=True)
        return 0

    lax.fori_loop(0, NCH // 2, step2, 0)
    if NCH % 2:
        c = NCH - 1
        cur_rows, cur_sem = bufs[c % 2]
        pltpu.make_async_copy(feat_hbm.at[src_v.at[c]], cur_rows,
                              cur_sem).wait()
        pltpu.sync_copy(cur_rows, agg_sh.at[dst_v.at[c]], add=True)
    plsc.subcore_barrier()

    pltpu.sync_copy(agg_sh.at[pl.ds(sid * STRIPE, STRIPE)], zbuf)
    pltpu.sync_copy(zbuf, out_hbm.at[cid, pl.ds(sid * STRIPE, STRIPE)])

    @pl.when(sid == 0)
    def _():
        pltpu.sync_copy(agg_sh.at[pl.ds(NTILE * STRIPE, TAIL)],
                        zbuf.at[pl.ds(0, TAIL)])
        pltpu.sync_copy(zbuf.at[pl.ds(0, TAIL)],
                        out_hbm.at[cid, pl.ds(NTILE * STRIPE, TAIL)])


@functools.lru_cache(maxsize=1)
def _sc_seg_sum():
    return pl.kernel(
        _sc_body,
        out_type=jax.ShapeDtypeStruct((2, N, H), jnp.float32),
        mesh=plsc.VectorSubcoreMesh(core_axis_name="c", subcore_axis_name="s"),
        scratch_types=[
            pltpu.VMEM((NCH, CHUNK), jnp.int32),
            pltpu.VMEM((NCH, CHUNK), jnp.int32),
            pltpu.VMEM((CHUNK, H), jnp.float32),
            pltpu.VMEM((CHUNK, H), jnp.float32),
            pltpu.VMEM((STRIPE, H), jnp.float32),
            pltpu.VMEM_SHARED((N, H), jnp.float32),
            pltpu.SemaphoreType.DMA,
            pltpu.SemaphoreType.DMA,
        ],
        compiler_params=pltpu.CompilerParams(use_tc_tiling_on_sc=False),
    )


def kernel(x, edge_index, batch, W0, b0, g0, be0, L0W, L0b, W1, b1, g1, be1,
           L1W, L1b):
    s = jnp.float32(1.0) / jnp.sqrt(jnp.float32(1.0 + 1e-5))
    a0 = (g0 * s).reshape(1, H)
    c0 = (be0 + b0 * g0 * s).reshape(1, H)
    a1 = (g1 * s).reshape(1, H)
    c1 = (be1 + b1 * g1 * s).reshape(1, H)
    l0 = L0W
    l1 = L1W
    l0b = L0b.reshape(1, 1)
    l1b = L1b.reshape(1, 1)
    batch2 = batch.reshape(N, 1)

    feat, np0, wsi0 = _tc1(x, batch2, W0, a0, c0, l0, l0b)

    srcr = edge_index[0].reshape(NW, NCH, CHUNK)
    dstr = edge_index[1].reshape(NW, NCH, CHUNK)
    zer = jnp.zeros((STRIPE, H), jnp.float32)
    agg2 = _sc_seg_sum()(feat, srcr, dstr, zer)

    npred, wsi = _tc2(feat, agg2, batch2, np0, wsi0, W1, a1, c1, l1, l1b)
    return (wsi.reshape(B, 1), npred)


# speedup vs baseline: 9.3454x; 1.0340x over previous
import functools

import jax
import jax.numpy as jnp
from jax import lax
from jax.experimental import pallas as pl
from jax.experimental.pallas import tpu as pltpu
from jax.experimental.pallas import tpu_sc as plsc

N = 10000
E = 320000
DF = 128
H = 64
B = 8

BLK = 1000
NBLK = N // BLK

NW = 32
EPW = E // NW
CHUNK = 80
NCH = EPW // CHUNK
NTILE = 16
STRIPE = 624
TAIL = N - NTILE * STRIPE


def _tc_feat_body(x_ref, w0_ref, a0_ref, c0_ref, feat_ref):
    h = jnp.dot(x_ref[...].astype(jnp.bfloat16),
                w0_ref[...].astype(jnp.bfloat16),
                preferred_element_type=jnp.float32)
    feat_ref[...] = jnp.maximum(h * a0_ref[...] + c0_ref[...], 0.0)


def _make_tc_feat(interpret=False):
    return pl.pallas_call(
        _tc_feat_body,
        interpret=interpret,
        grid=(NBLK,),
        in_specs=[
            pl.BlockSpec((BLK, DF), lambda i: (i, 0)),
            pl.BlockSpec((DF, H), lambda i: (0, 0)),
            pl.BlockSpec((1, H), lambda i: (0, 0)),
            pl.BlockSpec((1, H), lambda i: (0, 0)),
        ],
        out_specs=pl.BlockSpec((BLK, H), lambda i: (i, 0)),
        out_shape=jax.ShapeDtypeStruct((N, H), jnp.float32),
    )


def _tc_head_body(feat_ref, batch_ref, l0_ref, l0b_ref, np0_ref, wsi_ref):
    i = pl.program_id(0)
    np0 = jnp.dot(feat_ref[...].astype(jnp.bfloat16),
                  l0_ref[...].astype(jnp.bfloat16),
                  preferred_element_type=jnp.float32) + l0b_ref[0, 0]
    np0_ref[...] = np0
    seg = lax.broadcasted_iota(jnp.int32, (BLK, B), 1)
    vals = jnp.where(batch_ref[...] == seg, np0, -jnp.inf)
    m8 = jnp.max(vals, axis=0).reshape(1, B)
    prev = jnp.where(i == 0, jnp.full((1, B), -jnp.inf), wsi_ref[...])
    wsi_ref[...] = jnp.maximum(prev, m8)


def _make_tc_head(interpret=False):
    return pl.pallas_call(
        _tc_head_body,
        interpret=interpret,
        grid=(NBLK,),
        in_specs=[
            pl.BlockSpec((BLK, H), lambda i: (i, 0)),
            pl.BlockSpec((BLK, 1), lambda i: (i, 0)),
            pl.BlockSpec((H, 1), lambda i: (0, 0)),
            pl.BlockSpec(memory_space=pltpu.SMEM),
        ],
        out_specs=[
            pl.BlockSpec((BLK, 1), lambda i: (i, 0)),
            pl.BlockSpec((1, B), lambda i: (0, 0)),
        ],
        out_shape=[
            jax.ShapeDtypeStruct((N, 1), jnp.float32),
            jax.ShapeDtypeStruct((1, B), jnp.float32),
        ],
    )


def _tc2_body(feat_ref, agg_ref, batch_ref, np0_ref, wsi0_ref, w1_ref,
              a1_ref, c1_ref, l1_ref, l1b_ref, np_ref, wsi_ref):
    i = pl.program_id(0)
    h = feat_ref[...] + agg_ref[0] + agg_ref[1]
    z = jnp.dot(h.astype(jnp.bfloat16), w1_ref[...].astype(jnp.bfloat16),
                preferred_element_type=jnp.float32)
    feat2 = jnp.maximum(z * a1_ref[...] + c1_ref[...], 0.0)
    np1 = jnp.dot(feat2.astype(jnp.bfloat16), l1_ref[...].astype(jnp.bfloat16),
                  preferred_element_type=jnp.float32) + l1b_ref[0, 0]
    np_ref[...] = np0_ref[...] + np1
    seg = lax.broadcasted_iota(jnp.int32, (BLK, B), 1)
    vals = jnp.where(batch_ref[...] == seg, np1, -jnp.inf)
    m8 = jnp.max(vals, axis=0).reshape(1, B)
    prev = jnp.where(i == 0, jnp.full((1, B), -jnp.inf), wsi_ref[...])
    acc = jnp.maximum(prev, m8)
    wsi_ref[...] = acc

    @pl.when(i == NBLK - 1)
    def _():
        wsi_ref[...] = acc + wsi0_ref[...]


def _make_tc2(interpret=False):
    return pl.pallas_call(
        _tc2_body,
        interpret=interpret,
        grid=(NBLK,),
    in_specs=[
        pl.BlockSpec((BLK, H), lambda i: (i, 0)),
        pl.BlockSpec((2, BLK, H), lambda i: (0, i, 0)),
        pl.BlockSpec((BLK, 1), lambda i: (i, 0)),
        pl.BlockSpec((BLK, 1), lambda i: (i, 0)),
        pl.BlockSpec((1, B), lambda i: (0, 0)),
        pl.BlockSpec((H, H), lambda i: (0, 0)),
        pl.BlockSpec((1, H), lambda i: (0, 0)),
        pl.BlockSpec((1, H), lambda i: (0, 0)),
        pl.BlockSpec((H, 1), lambda i: (0, 0)),
        pl.BlockSpec(memory_space=pltpu.SMEM),
    ],
    out_specs=[
        pl.BlockSpec((BLK, 1), lambda i: (i, 0)),
        pl.BlockSpec((1, B), lambda i: (0, 0)),
    ],
    out_shape=[
        jax.ShapeDtypeStruct((N, 1), jnp.float32),
        jax.ShapeDtypeStruct((1, B), jnp.float32),
    ],
    )


_tc_feat = _make_tc_feat()
_tc_head = _make_tc_head()
_tc2 = _make_tc2()


def _sc_body(feat_hbm, src_hbm, dst_hbm, zer_hbm, out_hbm,
             src_v, dst_v, rows_a, rows_b, zbuf, agg_sh, sem_a, sem_b):
    cid = lax.axis_index("c")
    sid = lax.axis_index("s")
    wid = sid * 2 + cid

    pltpu.sync_copy(zer_hbm, zbuf)
    pltpu.sync_copy(zbuf, agg_sh.at[pl.ds(sid * STRIPE, STRIPE)])

    @pl.when(sid == 0)
    def _():
        pltpu.sync_copy(zbuf.at[pl.ds(0, TAIL)],
                        agg_sh.at[pl.ds(NTILE * STRIPE, TAIL)])

    plsc.subcore_barrier()

    pltpu.sync_copy(src_hbm.at[wid], src_v)
    pltpu.sync_copy(dst_hbm.at[wid], dst_v)

    bufs = ((rows_a, sem_a), (rows_b, sem_b))
    pltpu.async_copy(feat_hbm.at[src_v.at[0]], rows_a, sem_a)

    def step2(k, _):
        for b in range(2):
            c = 2 * k + b
            cur_rows, cur_sem = bufs[b]
            nxt_rows, nxt_sem = bufs[1 - b]
            pltpu.make_async_copy(feat_hbm.at[src_v.at[c]], cur_rows,
                                  cur_sem).wait()

            @pl.when(c + 1 < NCH)
            def _():
                pltpu.async_copy(feat_hbm.at[src_v.at[c + 1]], nxt_rows,
                                 nxt_sem)

            pltpu.sync_copy(cur_rows, agg_sh.at[dst_v.at[c]], add=True)
        return 0

    lax.fori_loop(0, NCH // 2, step2, 0)
    if NCH % 2:
        c = NCH - 1
        cur_rows, cur_sem = bufs[c % 2]
        pltpu.make_async_copy(feat_hbm.at[src_v.at[c]], cur_rows,
                              cur_sem).wait()
        pltpu.sync_copy(cur_rows, agg_sh.at[dst_v.at[c]], add=True)
    plsc.subcore_barrier()

    pltpu.sync_copy(agg_sh.at[pl.ds(sid * STRIPE, STRIPE)], zbuf)
    pltpu.sync_copy(zbuf, out_hbm.at[cid, pl.ds(sid * STRIPE, STRIPE)])

    @pl.when(sid == 0)
    def _():
        pltpu.sync_copy(agg_sh.at[pl.ds(NTILE * STRIPE, TAIL)],
                        zbuf.at[pl.ds(0, TAIL)])
        pltpu.sync_copy(zbuf.at[pl.ds(0, TAIL)],
                        out_hbm.at[cid, pl.ds(NTILE * STRIPE, TAIL)])


@functools.lru_cache(maxsize=1)
def _sc_seg_sum():
    return pl.kernel(
        _sc_body,
        out_type=jax.ShapeDtypeStruct((2, N, H), jnp.float32),
        mesh=plsc.VectorSubcoreMesh(core_axis_name="c", subcore_axis_name="s"),
        scratch_types=[
            pltpu.VMEM((NCH, CHUNK), jnp.int32),
            pltpu.VMEM((NCH, CHUNK), jnp.int32),
            pltpu.VMEM((CHUNK, H), jnp.float32),
            pltpu.VMEM((CHUNK, H), jnp.float32),
            pltpu.VMEM((STRIPE, H), jnp.float32),
            pltpu.VMEM_SHARED((N, H), jnp.float32),
            pltpu.SemaphoreType.DMA,
            pltpu.SemaphoreType.DMA,
        ],
        compiler_params=pltpu.CompilerParams(use_tc_tiling_on_sc=False),
    )


def kernel(x, edge_index, batch, W0, b0, g0, be0, L0W, L0b, W1, b1, g1, be1,
           L1W, L1b):
    s = jnp.float32(1.0) / jnp.sqrt(jnp.float32(1.0 + 1e-5))
    a0 = (g0 * s).reshape(1, H)
    c0 = (be0 + b0 * g0 * s).reshape(1, H)
    a1 = (g1 * s).reshape(1, H)
    c1 = (be1 + b1 * g1 * s).reshape(1, H)
    l0 = L0W
    l1 = L1W
    l0b = L0b.reshape(1, 1)
    l1b = L1b.reshape(1, 1)
    batch2 = batch.reshape(N, 1)

    feat = _tc_feat(x, W0, a0, c0)
    np0, wsi0 = _tc_head(feat, batch2, l0, l0b)

    srcr = edge_index[0].reshape(NW, NCH, CHUNK)
    dstr = edge_index[1].reshape(NW, NCH, CHUNK)
    zer = jnp.zeros((STRIPE, H), jnp.float32)
    agg2 = _sc_seg_sum()(feat, srcr, dstr, zer)

    npred, wsi = _tc2(feat, agg2, batch2, np0, wsi0, W1, a1, c1, l1, l1b)
    return (wsi.reshape(B, 1), npred)


# speedup vs baseline: 9.5379x; 1.0206x over previous
import functools

import jax
import jax.numpy as jnp
from jax import lax
from jax.experimental import pallas as pl
from jax.experimental.pallas import tpu as pltpu
from jax.experimental.pallas import tpu_sc as plsc

N = 10000
E = 320000
DF = 128
H = 64
B = 8

BLK = 2000
NBLK = N // BLK

NW = 32
EPW = E // NW
CHUNK = 80
NCH = EPW // CHUNK
NTILE = 16
STRIPE = 624
TAIL = N - NTILE * STRIPE


def _tc_feat_body(x_ref, w0_ref, a0_ref, c0_ref, feat_ref):
    h = jnp.dot(x_ref[...].astype(jnp.bfloat16),
                w0_ref[...].astype(jnp.bfloat16),
                preferred_element_type=jnp.float32)
    feat_ref[...] = jnp.maximum(h * a0_ref[...] + c0_ref[...], 0.0)


def _make_tc_feat(interpret=False):
    return pl.pallas_call(
        _tc_feat_body,
        interpret=interpret,
        grid=(NBLK,),
        in_specs=[
            pl.BlockSpec((BLK, DF), lambda i: (i, 0)),
            pl.BlockSpec((DF, H), lambda i: (0, 0)),
            pl.BlockSpec((1, H), lambda i: (0, 0)),
            pl.BlockSpec((1, H), lambda i: (0, 0)),
        ],
        out_specs=pl.BlockSpec((BLK, H), lambda i: (i, 0)),
        out_shape=jax.ShapeDtypeStruct((N, H), jnp.float32),
    )


def _tc_head_body(feat_ref, batch_ref, l0_ref, l0b_ref, np0_ref, wsi_ref):
    i = pl.program_id(0)
    np0 = jnp.dot(feat_ref[...].astype(jnp.bfloat16),
                  l0_ref[...].astype(jnp.bfloat16),
                  preferred_element_type=jnp.float32) + l0b_ref[0, 0]
    np0_ref[...] = np0
    seg = lax.broadcasted_iota(jnp.int32, (BLK, B), 1)
    vals = jnp.where(batch_ref[...] == seg, np0, -jnp.inf)
    m8 = jnp.max(vals, axis=0).reshape(1, B)
    prev = jnp.where(i == 0, jnp.full((1, B), -jnp.inf), wsi_ref[...])
    wsi_ref[...] = jnp.maximum(prev, m8)


def _make_tc_head(interpret=False):
    return pl.pallas_call(
        _tc_head_body,
        interpret=interpret,
        grid=(NBLK,),
        in_specs=[
            pl.BlockSpec((BLK, H), lambda i: (i, 0)),
            pl.BlockSpec((BLK, 1), lambda i: (i, 0)),
            pl.BlockSpec((H, 1), lambda i: (0, 0)),
            pl.BlockSpec(memory_space=pltpu.SMEM),
        ],
        out_specs=[
            pl.BlockSpec((BLK, 1), lambda i: (i, 0)),
            pl.BlockSpec((1, B), lambda i: (0, 0)),
        ],
        out_shape=[
            jax.ShapeDtypeStruct((N, 1), jnp.float32),
            jax.ShapeDtypeStruct((1, B), jnp.float32),
        ],
    )


def _tc2_body(feat_ref, agg_ref, batch_ref, np0_ref, wsi0_ref, w1_ref,
              a1_ref, c1_ref, l1_ref, l1b_ref, np_ref, wsi_ref):
    i = pl.program_id(0)
    h = feat_ref[...] + agg_ref[0] + agg_ref[1]
    z = jnp.dot(h.astype(jnp.bfloat16), w1_ref[...].astype(jnp.bfloat16),
                preferred_element_type=jnp.float32)
    feat2 = jnp.maximum(z * a1_ref[...] + c1_ref[...], 0.0)
    np1 = jnp.dot(feat2.astype(jnp.bfloat16), l1_ref[...].astype(jnp.bfloat16),
                  preferred_element_type=jnp.float32) + l1b_ref[0, 0]
    np_ref[...] = np0_ref[...] + np1
    seg = lax.broadcasted_iota(jnp.int32, (BLK, B), 1)
    vals = jnp.where(batch_ref[...] == seg, np1, -jnp.inf)
    m8 = jnp.max(vals, axis=0).reshape(1, B)
    prev = jnp.where(i == 0, jnp.full((1, B), -jnp.inf), wsi_ref[...])
    acc = jnp.maximum(prev, m8)
    wsi_ref[...] = acc

    @pl.when(i == NBLK - 1)
    def _():
        wsi_ref[...] = acc + wsi0_ref[...]


def _make_tc2(interpret=False):
    return pl.pallas_call(
        _tc2_body,
        interpret=interpret,
        grid=(NBLK,),
    in_specs=[
        pl.BlockSpec((BLK, H), lambda i: (i, 0)),
        pl.BlockSpec((2, BLK, H), lambda i: (0, i, 0)),
        pl.BlockSpec((BLK, 1), lambda i: (i, 0)),
        pl.BlockSpec((BLK, 1), lambda i: (i, 0)),
        pl.BlockSpec((1, B), lambda i: (0, 0)),
        pl.BlockSpec((H, H), lambda i: (0, 0)),
        pl.BlockSpec((1, H), lambda i: (0, 0)),
        pl.BlockSpec((1, H), lambda i: (0, 0)),
        pl.BlockSpec((H, 1), lambda i: (0, 0)),
        pl.BlockSpec(memory_space=pltpu.SMEM),
    ],
    out_specs=[
        pl.BlockSpec((BLK, 1), lambda i: (i, 0)),
        pl.BlockSpec((1, B), lambda i: (0, 0)),
    ],
    out_shape=[
        jax.ShapeDtypeStruct((N, 1), jnp.float32),
        jax.ShapeDtypeStruct((1, B), jnp.float32),
    ],
    )


_tc_feat = _make_tc_feat()
_tc_head = _make_tc_head()
_tc2 = _make_tc2()


def _sc_body(feat_hbm, src_hbm, dst_hbm, zer_hbm, out_hbm,
             src_v, dst_v, rows_a, rows_b, zbuf, agg_sh, sem_a, sem_b):
    cid = lax.axis_index("c")
    sid = lax.axis_index("s")
    wid = sid * 2 + cid

    pltpu.sync_copy(zer_hbm, zbuf)
    pltpu.sync_copy(zbuf, agg_sh.at[pl.ds(sid * STRIPE, STRIPE)])

    @pl.when(sid == 0)
    def _():
        pltpu.sync_copy(zbuf.at[pl.ds(0, TAIL)],
                        agg_sh.at[pl.ds(NTILE * STRIPE, TAIL)])

    plsc.subcore_barrier()

    pltpu.sync_copy(src_hbm.at[wid], src_v)
    pltpu.sync_copy(dst_hbm.at[wid], dst_v)

    bufs = ((rows_a, sem_a), (rows_b, sem_b))
    pltpu.async_copy(feat_hbm.at[src_v.at[0]], rows_a, sem_a)

    def step2(k, _):
        for b in range(2):
            c = 2 * k + b
            cur_rows, cur_sem = bufs[b]
            nxt_rows, nxt_sem = bufs[1 - b]
            pltpu.make_async_copy(feat_hbm.at[src_v.at[c]], cur_rows,
                                  cur_sem).wait()

            @pl.when(c + 1 < NCH)
            def _():
                pltpu.async_copy(feat_hbm.at[src_v.at[c + 1]], nxt_rows,
                                 nxt_sem)

            pltpu.sync_copy(cur_rows, agg_sh.at[dst_v.at[c]], add=True)
        return 0

    lax.fori_loop(0, NCH // 2, step2, 0)
    if NCH % 2:
        c = NCH - 1
        cur_rows, cur_sem = bufs[c % 2]
        pltpu.make_async_copy(feat_hbm.at[src_v.at[c]], cur_rows,
                              cur_sem).wait()
        pltpu.sync_copy(cur_rows, agg_sh.at[dst_v.at[c]], add=True)
    plsc.subcore_barrier()

    pltpu.sync_copy(agg_sh.at[pl.ds(sid * STRIPE, STRIPE)], zbuf)
    pltpu.sync_copy(zbuf, out_hbm.at[cid, pl.ds(sid * STRIPE, STRIPE)])

    @pl.when(sid == 0)
    def _():
        pltpu.sync_copy(agg_sh.at[pl.ds(NTILE * STRIPE, TAIL)],
                        zbuf.at[pl.ds(0, TAIL)])
        pltpu.sync_copy(zbuf.at[pl.ds(0, TAIL)],
                        out_hbm.at[cid, pl.ds(NTILE * STRIPE, TAIL)])


@functools.lru_cache(maxsize=1)
def _sc_seg_sum():
    return pl.kernel(
        _sc_body,
        out_type=jax.ShapeDtypeStruct((2, N, H), jnp.float32),
        mesh=plsc.VectorSubcoreMesh(core_axis_name="c", subcore_axis_name="s"),
        scratch_types=[
            pltpu.VMEM((NCH, CHUNK), jnp.int32),
            pltpu.VMEM((NCH, CHUNK), jnp.int32),
            pltpu.VMEM((CHUNK, H), jnp.float32),
            pltpu.VMEM((CHUNK, H), jnp.float32),
            pltpu.VMEM((STRIPE, H), jnp.float32),
            pltpu.VMEM_SHARED((N, H), jnp.float32),
            pltpu.SemaphoreType.DMA,
            pltpu.SemaphoreType.DMA,
        ],
        compiler_params=pltpu.CompilerParams(use_tc_tiling_on_sc=False),
    )


def kernel(x, edge_index, batch, W0, b0, g0, be0, L0W, L0b, W1, b1, g1, be1,
           L1W, L1b):
    s = jnp.float32(1.0) / jnp.sqrt(jnp.float32(1.0 + 1e-5))
    a0 = (g0 * s).reshape(1, H)
    c0 = (be0 + b0 * g0 * s).reshape(1, H)
    a1 = (g1 * s).reshape(1, H)
    c1 = (be1 + b1 * g1 * s).reshape(1, H)
    l0 = L0W
    l1 = L1W
    l0b = L0b.reshape(1, 1)
    l1b = L1b.reshape(1, 1)
    batch2 = batch.reshape(N, 1)

    feat = _tc_feat(x, W0, a0, c0)
    np0, wsi0 = _tc_head(feat, batch2, l0, l0b)

    srcr = edge_index[0].reshape(NW, NCH, CHUNK)
    dstr = edge_index[1].reshape(NW, NCH, CHUNK)
    zer = jnp.zeros((STRIPE, H), jnp.float32)
    agg2 = _sc_seg_sum()(feat, srcr, dstr, zer)

    npred, wsi = _tc2(feat, agg2, batch2, np0, wsi0, W1, a1, c1, l1, l1b)
    return (wsi.reshape(B, 1), npred)
